# baseline (speedup 1.0000x reference)
import jax
import jax.numpy as jnp
from jax import lax
from jax.experimental import pallas as pl
from jax.experimental.pallas import tpu as pltpu
from jax.experimental.pallas import tpu_sc as plsc

NC = 2
NS = 16
NW = NC * NS
L = 16

N = 10000
E = 320000
D = 128
H1 = 16
H2 = 32

CH = 128
ETOT = E + N
SB = -(-ETOT // (NW * CH))
EPAD = NW * SB * CH
NPAD = 10240
PT = NPAD // NS

NB = 3

_MESH = plsc.VectorSubcoreMesh(
    core_axis_name="c", subcore_axis_name="s", num_cores=NC, num_subcores=NS
)

_SC_PARAMS = pltpu.CompilerParams(
    needs_layout_passes=False, use_tc_tiling_on_sc=False
)

_BCAST_DNUMS = lax.GatherDimensionNumbers(
    offset_dims=(), collapsed_slice_dims=(0,), start_index_map=(0,)
)


def _lane_bcast(vec, u):
    idx = jnp.full((L, 1), u, jnp.int32)
    return lax.gather(vec, idx, _BCAST_DNUMS, slice_sizes=(1,),
                      mode=lax.GatherScatterMode.PROMISE_IN_BOUNDS)


def _zero_acc_slice(zero_v, acc, s, H):
    def _zero(i, _):
        for q in range(H // L):
            zero_v[i, pl.ds(q * L, L)] = jnp.zeros((L,), jnp.float32)
        return 0

    lax.fori_loop(0, PT, _zero, 0)
    pltpu.sync_copy(zero_v, acc.at[pl.ds(s * PT, PT)])


def _ring_agg(H, tab_hbm, idxr_v, idxc_v, ew_v, dinv_v, grow, msg,
              gsem, ssem, acc):
    for b in range(NB):
        pltpu.async_copy(tab_hbm.at[idxr_v.at[b]], grow[b], gsem[b])

    def _iter(g, _):
        for b in range(NB):
            j = g * NB + b
            pltpu.make_async_copy(tab_hbm.at[idxr_v.at[0]],
                                  grow[b], gsem[b]).wait()

            @pl.when(j >= NB)
            def _():
                pltpu.make_async_copy(msg[b], acc.at[idxc_v.at[0]],
                                      ssem[b]).wait()

            def _scale(q, _):
                r16 = idxr_v[j, pl.ds(q * L, L)]
                c16 = idxc_v[j, pl.ds(q * L, L)]
                e16 = ew_v[j, pl.ds(q * L, L)]
                dr = plsc.load_gather(dinv_v, [r16])
                dc = plsc.load_gather(dinv_v, [c16])
                norm16 = dr * e16 * dc
                for u in range(L):
                    nb = _lane_bcast(norm16, u)
                    r = q * L + u
                    for hq in range(H // L):
                        msg[b][r, pl.ds(hq * L, L)] = (
                            grow[b][r, pl.ds(hq * L, L)] * nb
                        )
                return 0

            lax.fori_loop(0, CH // L, _scale, 0)

            @pl.when(j + NB < SB)
            def _():
                pltpu.async_copy(tab_hbm.at[idxr_v.at[j + NB]],
                                 grow[b], gsem[b])

            pltpu.async_copy(msg[b], acc.at[idxc_v.at[j]], ssem[b],
                             add=True)
        return 0

    lax.fori_loop(0, SB // NB, _iter, 0)
    for b in range(NB):
        pltpu.make_async_copy(msg[b], acc.at[idxc_v.at[0]], ssem[b]).wait()


def _agg1_body(tab_hbm, row_hbm, col_hbm, ew_hbm, part_hbm, dinv_hbm,
               dinv_v, idxr_v, idxc_v, ew_v, dcol_v, dew_v, degs_v,
               g0, g1, g2, m0, m1, m2, zero_v, acc1, acc,
               gs0, gs1, gs2, ss0, ss1, ss2, dsem):
    grow = (g0, g1, g2)
    msg = (m0, m1, m2)
    gsem = (gs0, gs1, gs2)
    ssem = (ss0, ss1, ss2)
    c = lax.axis_index("c")
    s = lax.axis_index("s")
    w = c * NS + s

    pltpu.sync_copy(row_hbm.at[w], idxr_v)
    pltpu.sync_copy(col_hbm.at[w], idxc_v)
    pltpu.sync_copy(ew_hbm.at[w], ew_v)

    def _zd(i, _):
        degs_v[pl.ds(i * L, L)] = jnp.zeros((L,), jnp.float32)
        return 0

    lax.fori_loop(0, PT // L, _zd, 0)
    pltpu.sync_copy(degs_v, acc1.at[pl.ds(s * PT, PT)])
    _zero_acc_slice(zero_v, acc, s, H1)
    plsc.subcore_barrier()

    for k in range(NC):
        wd = k * NS + s
        pltpu.sync_copy(col_hbm.at[wd], dcol_v)
        pltpu.sync_copy(ew_hbm.at[wd], dew_v)

        def _dstep(j, _):
            pltpu.async_copy(dew_v.at[j], acc1.at[dcol_v.at[j]], dsem,
                             add=True)
            return 0

        lax.fori_loop(0, SB, _dstep, 0)

        def _dwait(j, _):
            pltpu.make_async_copy(dew_v.at[0], acc1.at[dcol_v.at[0]],
                                  dsem).wait()
            return 0

        lax.fori_loop(0, SB, _dwait, 0)
    plsc.subcore_barrier()

    pltpu.sync_copy(acc1.at[pl.ds(s * PT, PT)], degs_v)

    def _newton(k, _):
        d = degs_v[pl.ds(k * L, L)]
        i = plsc.bitcast(d, jnp.int32)
        i = jnp.int32(0x5F3759DF) - (i >> 1)
        y = plsc.bitcast(i, jnp.float32)
        h = d * 0.5
        y = y * (1.5 - h * y * y)
        y = y * (1.5 - h * y * y)
        y = y * (1.5 - h * y * y)
        y = jnp.where(d > 0.0, y, 0.0)
        degs_v[pl.ds(k * L, L)] = y
        return 0

    lax.fori_loop(0, PT // L, _newton, 0)
    pltpu.sync_copy(degs_v, acc1.at[pl.ds(s * PT, PT)])

    @pl.when(c == 0)
    def _():
        pltpu.sync_copy(degs_v, dinv_hbm.at[pl.ds(s * PT, PT)])

    plsc.subcore_barrier()
    pltpu.sync_copy(acc1, dinv_v)

    _ring_agg(H1, tab_hbm, idxr_v, idxc_v, ew_v, dinv_v, grow, msg,
              gsem, ssem, acc)
    plsc.subcore_barrier()
    pltpu.sync_copy(acc.at[pl.ds(s * PT, PT)],
                    part_hbm.at[c, pl.ds(s * PT, PT)])


_agg1_call = pl.kernel(
    _agg1_body,
    out_type=(
        jax.ShapeDtypeStruct((NC, NPAD, H1), jnp.float32),
        jax.ShapeDtypeStruct((NPAD,), jnp.float32),
    ),
    mesh=_MESH,
    compiler_params=_SC_PARAMS,
    scratch_types=[
        pltpu.VMEM((NPAD,), jnp.float32),
        pltpu.VMEM((SB, CH), jnp.int32),
        pltpu.VMEM((SB, CH), jnp.int32),
        pltpu.VMEM((SB, CH), jnp.float32),
        pltpu.VMEM((SB, CH), jnp.int32),
        pltpu.VMEM((SB, CH), jnp.float32),
        pltpu.VMEM((PT,), jnp.float32),
        pltpu.VMEM((CH, H1), jnp.float32),
        pltpu.VMEM((CH, H1), jnp.float32),
        pltpu.VMEM((CH, H1), jnp.float32),
        pltpu.VMEM((CH, H1), jnp.float32),
        pltpu.VMEM((CH, H1), jnp.float32),
        pltpu.VMEM((CH, H1), jnp.float32),
        pltpu.VMEM((PT, H1), jnp.float32),
        pltpu.VMEM_SHARED((NPAD,), jnp.float32),
        pltpu.VMEM_SHARED((NPAD, H1), jnp.float32),
        pltpu.SemaphoreType.DMA,
        pltpu.SemaphoreType.DMA,
        pltpu.SemaphoreType.DMA,
        pltpu.SemaphoreType.DMA,
        pltpu.SemaphoreType.DMA,
        pltpu.SemaphoreType.DMA,
        pltpu.SemaphoreType.DMA,
    ],
)


def _agg2_body(tab_hbm, dinv_hbm, row_hbm, col_hbm, ew_hbm, out_hbm,
               dinv_v, idxr_v, idxc_v, ew_v,
               g0, g1, g2, m0, m1, m2, zero_v, acc,
               gs0, gs1, gs2, ss0, ss1, ss2):
    grow = (g0, g1, g2)
    msg = (m0, m1, m2)
    gsem = (gs0, gs1, gs2)
    ssem = (ss0, ss1, ss2)
    c = lax.axis_index("c")
    s = lax.axis_index("s")
    w = c * NS + s

    pltpu.sync_copy(dinv_hbm, dinv_v)
    pltpu.sync_copy(row_hbm.at[w], idxr_v)
    pltpu.sync_copy(col_hbm.at[w], idxc_v)
    pltpu.sync_copy(ew_hbm.at[w], ew_v)
    _zero_acc_slice(zero_v, acc, s, H2)
    plsc.subcore_barrier()

    _ring_agg(H2, tab_hbm, idxr_v, idxc_v, ew_v, dinv_v, grow, msg,
              gsem, ssem, acc)
    plsc.subcore_barrier()
    pltpu.sync_copy(acc.at[pl.ds(s * PT, PT)],
                    out_hbm.at[c, pl.ds(s * PT, PT)])


_agg2_call = pl.kernel(
    _agg2_body,
    out_type=jax.ShapeDtypeStruct((NC, NPAD, H2), jnp.float32),
    mesh=_MESH,
    compiler_params=_SC_PARAMS,
    scratch_types=[
        pltpu.VMEM((NPAD,), jnp.float32),
        pltpu.VMEM((SB, CH), jnp.int32),
        pltpu.VMEM((SB, CH), jnp.int32),
        pltpu.VMEM((SB, CH), jnp.float32),
        pltpu.VMEM((CH, H2), jnp.float32),
        pltpu.VMEM((CH, H2), jnp.float32),
        pltpu.VMEM((CH, H2), jnp.float32),
        pltpu.VMEM((CH, H2), jnp.float32),
        pltpu.VMEM((CH, H2), jnp.float32),
        pltpu.VMEM((CH, H2), jnp.float32),
        pltpu.VMEM((PT, H2), jnp.float32),
        pltpu.VMEM_SHARED((NPAD, H2), jnp.float32),
        pltpu.SemaphoreType.DMA,
        pltpu.SemaphoreType.DMA,
        pltpu.SemaphoreType.DMA,
        pltpu.SemaphoreType.DMA,
        pltpu.SemaphoreType.DMA,
        pltpu.SemaphoreType.DMA,
    ],
)


def _lin1_body(x_ref, w_ref, o_ref):
    o_ref[...] = jnp.dot(x_ref[...], w_ref[...],
                         preferred_element_type=jnp.float32)


def _relu_lin_body(p_ref, b_ref, w2_ref, o_ref):
    h = jnp.maximum(p_ref[0] + p_ref[1] + b_ref[...], 0.0)
    o_ref[...] = jnp.dot(h, w2_ref[...], preferred_element_type=jnp.float32)


def _head_body(q_ref, b2_ref, wfc_ref, bfc_ref, o_ref):
    m = q_ref[0] + q_ref[1] + b2_ref[...]
    o_ref[...] = jnp.dot(m, wfc_ref[...],
                         preferred_element_type=jnp.float32) + bfc_ref[...]


def kernel(x, edge_index, edge_weight, W1, b1, W2, b2, Wfc, bfc):
    n = x.shape[0]
    loop = jnp.arange(n, dtype=edge_index.dtype)
    row = jnp.concatenate([edge_index[0], loop])
    col = jnp.concatenate([edge_index[1], loop])
    ew = jnp.concatenate([edge_weight, jnp.ones((n,), edge_weight.dtype)])
    pad = EPAD - ETOT
    row3 = jnp.pad(row, (0, pad)).reshape(NW, SB, CH).astype(jnp.int32)
    col3 = jnp.pad(col, (0, pad)).reshape(NW, SB, CH).astype(jnp.int32)
    ew3 = jnp.pad(ew, (0, pad)).reshape(NW, SB, CH)
    xpad = jnp.pad(x, ((0, NPAD - n), (0, 0)))

    xw1 = pl.pallas_call(
        _lin1_body,
        out_shape=jax.ShapeDtypeStruct((NPAD, H1), jnp.float32),
    )(xpad, W1)

    p1, dinv = _agg1_call(xw1, row3, col3, ew3)

    xw2 = pl.pallas_call(
        _relu_lin_body,
        out_shape=jax.ShapeDtypeStruct((NPAD, H2), jnp.float32),
    )(p1, b1.reshape(1, H1), W2)

    p2 = _agg2_call(xw2, dinv, row3, col3, ew3)

    out = pl.pallas_call(
        _head_body,
        out_shape=jax.ShapeDtypeStruct((NPAD, 1), jnp.float32),
    )(p2, b2.reshape(1, H2), Wfc, bfc.reshape(1, 1))
    return out[:n]


# speedup vs baseline: 1.0035x; 1.0035x over previous
import jax
import jax.numpy as jnp
from jax import lax
from jax.experimental import pallas as pl
from jax.experimental.pallas import tpu as pltpu
from jax.experimental.pallas import tpu_sc as plsc

NC = 2
NS = 16
NW = NC * NS
L = 16

N = 10000
E = 320000
D = 128
H1 = 16
H2 = 32

CH = 128
ETOT = E + N
SB = -(-ETOT // (NW * CH))
EPAD = NW * SB * CH
NPAD = 10240
PT = NPAD // NS

NB = 3

_MESH = plsc.VectorSubcoreMesh(
    core_axis_name="c", subcore_axis_name="s", num_cores=NC, num_subcores=NS
)

_SC_PARAMS = pltpu.CompilerParams(
    needs_layout_passes=False, use_tc_tiling_on_sc=False
)

_BCAST_DNUMS = lax.GatherDimensionNumbers(
    offset_dims=(), collapsed_slice_dims=(0,), start_index_map=(0,)
)


def _lane_bcast(vec, u):
    idx = jnp.full((L, 1), u, jnp.int32)
    return lax.gather(vec, idx, _BCAST_DNUMS, slice_sizes=(1,),
                      mode=lax.GatherScatterMode.PROMISE_IN_BOUNDS)


def _zero_acc_slice(zero_v, acc, s, H):
    def _zero(i, _):
        for q in range(H // L):
            zero_v[i, pl.ds(q * L, L)] = jnp.zeros((L,), jnp.float32)
        return 0

    lax.fori_loop(0, PT, _zero, 0)
    pltpu.sync_copy(zero_v, acc.at[pl.ds(s * PT, PT)])


def _ring_agg(H, tab_hbm, idxr_v, idxc_v, ew_v, dinv_v, grow, msg,
              gsem, ssem, acc):
    for b in range(NB):
        pltpu.async_copy(tab_hbm.at[idxr_v.at[b]], grow[b], gsem[b])

    def _iter(g, _):
        for b in range(NB):
            j = g * NB + b
            pltpu.make_async_copy(tab_hbm.at[idxr_v.at[0]],
                                  grow[b], gsem[b]).wait()

            @pl.when(j >= NB)
            def _():
                pltpu.make_async_copy(msg[b], acc.at[idxc_v.at[0]],
                                      ssem[b]).wait()

            def _scale(q, _):
                r16 = idxr_v[j, pl.ds(q * L, L)]
                c16 = idxc_v[j, pl.ds(q * L, L)]
                e16 = ew_v[j, pl.ds(q * L, L)]
                dr = plsc.load_gather(dinv_v, [r16])
                dc = plsc.load_gather(dinv_v, [c16])
                norm16 = dr * e16 * dc
                for u in range(L):
                    nb = _lane_bcast(norm16, u)
                    r = q * L + u
                    for hq in range(H // L):
                        msg[b][r, pl.ds(hq * L, L)] = (
                            grow[b][r, pl.ds(hq * L, L)] * nb
                        )
                return 0

            lax.fori_loop(0, CH // L, _scale, 0)

            @pl.when(j + NB < SB)
            def _():
                pltpu.async_copy(tab_hbm.at[idxr_v.at[j + NB]],
                                 grow[b], gsem[b])

            pltpu.async_copy(msg[b], acc.at[idxc_v.at[j]], ssem[b],
                             add=True)
        return 0

    lax.fori_loop(0, SB // NB, _iter, 0)
    for b in range(NB):
        pltpu.make_async_copy(msg[b], acc.at[idxc_v.at[0]], ssem[b]).wait()


def _agg1_body(tab_hbm, row_hbm, col_hbm, ew_hbm, part_hbm, dinv_hbm,
               dinv_v, idxr_v, idxc_v, ew_v, dcol_v, dew_v, degs_v,
               g0, g1, g2, m0, m1, m2, zero_v, acc1, acc,
               gs0, gs1, gs2, ss0, ss1, ss2, dsem):
    grow = (g0, g1, g2)
    msg = (m0, m1, m2)
    gsem = (gs0, gs1, gs2)
    ssem = (ss0, ss1, ss2)
    c = lax.axis_index("c")
    s = lax.axis_index("s")
    w = c * NS + s

    pltpu.sync_copy(row_hbm.at[w], idxr_v)
    pltpu.sync_copy(col_hbm.at[w], idxc_v)
    pltpu.sync_copy(ew_hbm.at[w], ew_v)

    def _zd(i, _):
        degs_v[pl.ds(i * L, L)] = jnp.zeros((L,), jnp.float32)
        return 0

    lax.fori_loop(0, PT // L, _zd, 0)
    pltpu.sync_copy(degs_v, acc1.at[pl.ds(s * PT, PT)])
    _zero_acc_slice(zero_v, acc, s, H1)
    plsc.subcore_barrier()

    for k in range(NC):
        wd = k * NS + s
        pltpu.sync_copy(col_hbm.at[wd], dcol_v)
        pltpu.sync_copy(ew_hbm.at[wd], dew_v)

        def _dstep(j, _):
            pltpu.async_copy(dew_v.at[j], acc1.at[dcol_v.at[j]], dsem,
                             add=True)
            return 0

        lax.fori_loop(0, SB, _dstep, 0)

        def _dwait(j, _):
            pltpu.make_async_copy(dew_v.at[0], acc1.at[dcol_v.at[0]],
                                  dsem).wait()
            return 0

        lax.fori_loop(0, SB, _dwait, 0)
    plsc.subcore_barrier()

    pltpu.sync_copy(acc1.at[pl.ds(s * PT, PT)], degs_v)

    def _newton(k, _):
        d = degs_v[pl.ds(k * L, L)]
        i = plsc.bitcast(d, jnp.int32)
        i = jnp.int32(0x5F3759DF) - (i >> 1)
        y = plsc.bitcast(i, jnp.float32)
        h = d * 0.5
        y = y * (1.5 - h * y * y)
        y = y * (1.5 - h * y * y)
        y = y * (1.5 - h * y * y)
        y = jnp.where(d > 0.0, y, 0.0)
        degs_v[pl.ds(k * L, L)] = y
        return 0

    lax.fori_loop(0, PT // L, _newton, 0)
    pltpu.sync_copy(degs_v, acc1.at[pl.ds(s * PT, PT)])

    @pl.when(c == 0)
    def _():
        pltpu.sync_copy(degs_v, dinv_hbm.at[pl.ds(s * PT, PT)])

    plsc.subcore_barrier()
    pltpu.sync_copy(acc1, dinv_v)

    _ring_agg(H1, tab_hbm, idxr_v, idxc_v, ew_v, dinv_v, grow, msg,
              gsem, ssem, acc)
    plsc.subcore_barrier()
    pltpu.sync_copy(acc.at[pl.ds(s * PT, PT)],
                    part_hbm.at[c, pl.ds(s * PT, PT)])


_agg1_call = pl.kernel(
    _agg1_body,
    out_type=(
        jax.ShapeDtypeStruct((NC, NPAD, H1), jnp.float32),
        jax.ShapeDtypeStruct((NPAD,), jnp.float32),
    ),
    mesh=_MESH,
    compiler_params=_SC_PARAMS,
    scratch_types=[
        pltpu.VMEM((NPAD,), jnp.float32),
        pltpu.VMEM((SB, CH), jnp.int32),
        pltpu.VMEM((SB, CH), jnp.int32),
        pltpu.VMEM((SB, CH), jnp.float32),
        pltpu.VMEM((SB, CH), jnp.int32),
        pltpu.VMEM((SB, CH), jnp.float32),
        pltpu.VMEM((PT,), jnp.float32),
        pltpu.VMEM((CH, H1), jnp.float32),
        pltpu.VMEM((CH, H1), jnp.float32),
        pltpu.VMEM((CH, H1), jnp.float32),
        pltpu.VMEM((CH, H1), jnp.float32),
        pltpu.VMEM((CH, H1), jnp.float32),
        pltpu.VMEM((CH, H1), jnp.float32),
        pltpu.VMEM((PT, H1), jnp.float32),
        pltpu.VMEM_SHARED((NPAD,), jnp.float32),
        pltpu.VMEM_SHARED((NPAD, H1), jnp.float32),
        pltpu.SemaphoreType.DMA,
        pltpu.SemaphoreType.DMA,
        pltpu.SemaphoreType.DMA,
        pltpu.SemaphoreType.DMA,
        pltpu.SemaphoreType.DMA,
        pltpu.SemaphoreType.DMA,
        pltpu.SemaphoreType.DMA,
    ],
)


def _agg2_body(tab_hbm, dinv_hbm, row_hbm, col_hbm, ew_hbm, out_hbm,
               dinv_v, idxr_v, idxc_v, ew_v,
               g0, g1, g2, m0, m1, m2, zero_v, tab_s, acc,
               gs0, gs1, gs2, ss0, ss1, ss2):
    grow = (g0, g1, g2)
    msg = (m0, m1, m2)
    gsem = (gs0, gs1, gs2)
    ssem = (ss0, ss1, ss2)
    c = lax.axis_index("c")
    s = lax.axis_index("s")
    w = c * NS + s

    pltpu.sync_copy(dinv_hbm, dinv_v)
    pltpu.sync_copy(row_hbm.at[w], idxr_v)
    pltpu.sync_copy(col_hbm.at[w], idxc_v)
    pltpu.sync_copy(ew_hbm.at[w], ew_v)
    pltpu.sync_copy(tab_hbm.at[pl.ds(s * PT, PT)],
                    tab_s.at[pl.ds(s * PT, PT)])
    _zero_acc_slice(zero_v, acc, s, H2)
    plsc.subcore_barrier()

    _ring_agg(H2, tab_s, idxr_v, idxc_v, ew_v, dinv_v, grow, msg,
              gsem, ssem, acc)
    plsc.subcore_barrier()
    pltpu.sync_copy(acc.at[pl.ds(s * PT, PT)],
                    out_hbm.at[c, pl.ds(s * PT, PT)])


_agg2_call = pl.kernel(
    _agg2_body,
    out_type=jax.ShapeDtypeStruct((NC, NPAD, H2), jnp.float32),
    mesh=_MESH,
    compiler_params=_SC_PARAMS,
    scratch_types=[
        pltpu.VMEM((NPAD,), jnp.float32),
        pltpu.VMEM((SB, CH), jnp.int32),
        pltpu.VMEM((SB, CH), jnp.int32),
        pltpu.VMEM((SB, CH), jnp.float32),
        pltpu.VMEM((CH, H2), jnp.float32),
        pltpu.VMEM((CH, H2), jnp.float32),
        pltpu.VMEM((CH, H2), jnp.float32),
        pltpu.VMEM((CH, H2), jnp.float32),
        pltpu.VMEM((CH, H2), jnp.float32),
        pltpu.VMEM((CH, H2), jnp.float32),
        pltpu.VMEM((PT, H2), jnp.float32),
        pltpu.VMEM_SHARED((NPAD, H2), jnp.float32),
        pltpu.VMEM_SHARED((NPAD, H2), jnp.float32),
        pltpu.SemaphoreType.DMA,
        pltpu.SemaphoreType.DMA,
        pltpu.SemaphoreType.DMA,
        pltpu.SemaphoreType.DMA,
        pltpu.SemaphoreType.DMA,
        pltpu.SemaphoreType.DMA,
    ],
)


def _lin1_body(x_ref, w_ref, o_ref):
    o_ref[...] = jnp.dot(x_ref[...], w_ref[...],
                         preferred_element_type=jnp.float32)


def _relu_lin_body(p_ref, b_ref, w2_ref, o_ref):
    h = jnp.maximum(p_ref[0] + p_ref[1] + b_ref[...], 0.0)
    o_ref[...] = jnp.dot(h, w2_ref[...], preferred_element_type=jnp.float32)


def _head_body(q_ref, b2_ref, wfc_ref, bfc_ref, o_ref):
    m = q_ref[0] + q_ref[1] + b2_ref[...]
    o_ref[...] = jnp.dot(m, wfc_ref[...],
                         preferred_element_type=jnp.float32) + bfc_ref[...]


def kernel(x, edge_index, edge_weight, W1, b1, W2, b2, Wfc, bfc):
    n = x.shape[0]
    loop = jnp.arange(n, dtype=edge_index.dtype)
    row = jnp.concatenate([edge_index[0], loop])
    col = jnp.concatenate([edge_index[1], loop])
    ew = jnp.concatenate([edge_weight, jnp.ones((n,), edge_weight.dtype)])
    pad = EPAD - ETOT
    row3 = jnp.pad(row, (0, pad)).reshape(NW, SB, CH).astype(jnp.int32)
    col3 = jnp.pad(col, (0, pad)).reshape(NW, SB, CH).astype(jnp.int32)
    ew3 = jnp.pad(ew, (0, pad)).reshape(NW, SB, CH)
    xpad = jnp.pad(x, ((0, NPAD - n), (0, 0)))

    xw1 = pl.pallas_call(
        _lin1_body,
        out_shape=jax.ShapeDtypeStruct((NPAD, H1), jnp.float32),
    )(xpad, W1)

    p1, dinv = _agg1_call(xw1, row3, col3, ew3)

    xw2 = pl.pallas_call(
        _relu_lin_body,
        out_shape=jax.ShapeDtypeStruct((NPAD, H2), jnp.float32),
    )(p1, b1.reshape(1, H1), W2)

    p2 = _agg2_call(xw2, dinv, row3, col3, ew3)

    out = pl.pallas_call(
        _head_body,
        out_shape=jax.ShapeDtypeStruct((NPAD, 1), jnp.float32),
    )(p2, b2.reshape(1, H2), Wfc, bfc.reshape(1, 1))
    return out[:n]


# speedup vs baseline: 1.4801x; 1.4750x over previous
import jax
import jax.numpy as jnp
from jax import lax
from jax.experimental import pallas as pl
from jax.experimental.pallas import tpu as pltpu
from jax.experimental.pallas import tpu_sc as plsc

NC = 2
NS = 16
NW = NC * NS
L = 16

N = 10000
E = 320000
D = 128
H1 = 16
H2 = 32

CH = 128
ETOT = E + N
SB = -(-ETOT // (NW * CH))
EPAD = NW * SB * CH
NPAD = 10240
PT = NPAD // NS

NB = 3

_MESH = plsc.VectorSubcoreMesh(
    core_axis_name="c", subcore_axis_name="s", num_cores=NC, num_subcores=NS
)

_SC_PARAMS = pltpu.CompilerParams(
    needs_layout_passes=False, use_tc_tiling_on_sc=False
)

_BCAST_DNUMS = lax.GatherDimensionNumbers(
    offset_dims=(), collapsed_slice_dims=(0,), start_index_map=(0,)
)


def _lane_bcast(vec, u):
    idx = jnp.full((L, 1), u, jnp.int32)
    return lax.gather(vec, idx, _BCAST_DNUMS, slice_sizes=(1,),
                      mode=lax.GatherScatterMode.PROMISE_IN_BOUNDS)


def _zero_acc_slice(zero_v, acc, s, H):
    def _zero(i, _):
        for q in range(H // L):
            zero_v[i, pl.ds(q * L, L)] = jnp.zeros((L,), jnp.float32)
        return 0

    lax.fori_loop(0, PT, _zero, 0)
    pltpu.sync_copy(zero_v, acc.at[pl.ds(s * PT, PT)])


def _ring_agg(H, tab_hbm, idxr_v, idxc_v, ew_v, dinv_v, grow, msg,
              gsem, ssem, acc):
    for b in range(NB):
        pltpu.async_copy(tab_hbm.at[idxr_v.at[b]], grow[b], gsem[b])

    def _iter(g, _):
        for b in range(NB):
            j = g * NB + b
            pltpu.make_async_copy(tab_hbm.at[idxr_v.at[0]],
                                  grow[b], gsem[b]).wait()

            @pl.when(j >= NB)
            def _():
                pltpu.make_async_copy(msg[b], acc.at[idxc_v.at[0]],
                                      ssem[b]).wait()

            for q in range(CH // L):
                r16 = idxr_v[j, pl.ds(q * L, L)]
                c16 = idxc_v[j, pl.ds(q * L, L)]
                e16 = ew_v[j, pl.ds(q * L, L)]
                dr = plsc.load_gather(dinv_v, [r16])
                dc = plsc.load_gather(dinv_v, [c16])
                norm16 = dr * e16 * dc
                for u in range(L):
                    nb = _lane_bcast(norm16, u)
                    r = q * L + u
                    for hq in range(H // L):
                        msg[b][r, pl.ds(hq * L, L)] = (
                            grow[b][r, pl.ds(hq * L, L)] * nb
                        )

            @pl.when(j + NB < SB)
            def _():
                pltpu.async_copy(tab_hbm.at[idxr_v.at[j + NB]],
                                 grow[b], gsem[b])

            pltpu.async_copy(msg[b], acc.at[idxc_v.at[j]], ssem[b],
                             add=True)
        return 0

    lax.fori_loop(0, SB // NB, _iter, 0)
    for b in range(NB):
        pltpu.make_async_copy(msg[b], acc.at[idxc_v.at[0]], ssem[b]).wait()


def _agg1_body(tab_hbm, row_hbm, col_hbm, ew_hbm, part_hbm, dinv_hbm,
               dinv_v, idxr_v, idxc_v, ew_v, dcol_v, dew_v, degs_v,
               g0, g1, g2, m0, m1, m2, zero_v, acc1, acc,
               gs0, gs1, gs2, ss0, ss1, ss2, dsem):
    grow = (g0, g1, g2)
    msg = (m0, m1, m2)
    gsem = (gs0, gs1, gs2)
    ssem = (ss0, ss1, ss2)
    c = lax.axis_index("c")
    s = lax.axis_index("s")
    w = c * NS + s

    pltpu.sync_copy(row_hbm.at[w], idxr_v)
    pltpu.sync_copy(col_hbm.at[w], idxc_v)
    pltpu.sync_copy(ew_hbm.at[w], ew_v)

    def _zd(i, _):
        degs_v[pl.ds(i * L, L)] = jnp.zeros((L,), jnp.float32)
        return 0

    lax.fori_loop(0, PT // L, _zd, 0)
    pltpu.sync_copy(degs_v, acc1.at[pl.ds(s * PT, PT)])
    _zero_acc_slice(zero_v, acc, s, H1)
    plsc.subcore_barrier()

    for k in range(NC):
        wd = k * NS + s
        pltpu.sync_copy(col_hbm.at[wd], dcol_v)
        pltpu.sync_copy(ew_hbm.at[wd], dew_v)

        def _dstep(j, _):
            pltpu.async_copy(dew_v.at[j], acc1.at[dcol_v.at[j]], dsem,
                             add=True)
            return 0

        lax.fori_loop(0, SB, _dstep, 0)

        def _dwait(j, _):
            pltpu.make_async_copy(dew_v.at[0], acc1.at[dcol_v.at[0]],
                                  dsem).wait()
            return 0

        lax.fori_loop(0, SB, _dwait, 0)
    plsc.subcore_barrier()

    pltpu.sync_copy(acc1.at[pl.ds(s * PT, PT)], degs_v)

    def _newton(k, _):
        d = degs_v[pl.ds(k * L, L)]
        i = plsc.bitcast(d, jnp.int32)
        i = jnp.int32(0x5F3759DF) - (i >> 1)
        y = plsc.bitcast(i, jnp.float32)
        h = d * 0.5
        y = y * (1.5 - h * y * y)
        y = y * (1.5 - h * y * y)
        y = y * (1.5 - h * y * y)
        y = jnp.where(d > 0.0, y, 0.0)
        degs_v[pl.ds(k * L, L)] = y
        return 0

    lax.fori_loop(0, PT // L, _newton, 0)
    pltpu.sync_copy(degs_v, acc1.at[pl.ds(s * PT, PT)])

    @pl.when(c == 0)
    def _():
        pltpu.sync_copy(degs_v, dinv_hbm.at[pl.ds(s * PT, PT)])

    plsc.subcore_barrier()
    pltpu.sync_copy(acc1, dinv_v)

    _ring_agg(H1, tab_hbm, idxr_v, idxc_v, ew_v, dinv_v, grow, msg,
              gsem, ssem, acc)
    plsc.subcore_barrier()
    pltpu.sync_copy(acc.at[pl.ds(s * PT, PT)],
                    part_hbm.at[c, pl.ds(s * PT, PT)])


_agg1_call = pl.kernel(
    _agg1_body,
    out_type=(
        jax.ShapeDtypeStruct((NC, NPAD, H1), jnp.float32),
        jax.ShapeDtypeStruct((NPAD,), jnp.float32),
    ),
    mesh=_MESH,
    compiler_params=_SC_PARAMS,
    scratch_types=[
        pltpu.VMEM((NPAD,), jnp.float32),
        pltpu.VMEM((SB, CH), jnp.int32),
        pltpu.VMEM((SB, CH), jnp.int32),
        pltpu.VMEM((SB, CH), jnp.float32),
        pltpu.VMEM((SB, CH), jnp.int32),
        pltpu.VMEM((SB, CH), jnp.float32),
        pltpu.VMEM((PT,), jnp.float32),
        pltpu.VMEM((CH, H1), jnp.float32),
        pltpu.VMEM((CH, H1), jnp.float32),
        pltpu.VMEM((CH, H1), jnp.float32),
        pltpu.VMEM((CH, H1), jnp.float32),
        pltpu.VMEM((CH, H1), jnp.float32),
        pltpu.VMEM((CH, H1), jnp.float32),
        pltpu.VMEM((PT, H1), jnp.float32),
        pltpu.VMEM_SHARED((NPAD,), jnp.float32),
        pltpu.VMEM_SHARED((NPAD, H1), jnp.float32),
        pltpu.SemaphoreType.DMA,
        pltpu.SemaphoreType.DMA,
        pltpu.SemaphoreType.DMA,
        pltpu.SemaphoreType.DMA,
        pltpu.SemaphoreType.DMA,
        pltpu.SemaphoreType.DMA,
        pltpu.SemaphoreType.DMA,
    ],
)


def _agg2_body(tab_hbm, dinv_hbm, row_hbm, col_hbm, ew_hbm, out_hbm,
               dinv_v, idxr_v, idxc_v, ew_v,
               g0, g1, g2, m0, m1, m2, zero_v, tab_s, acc,
               gs0, gs1, gs2, ss0, ss1, ss2):
    grow = (g0, g1, g2)
    msg = (m0, m1, m2)
    gsem = (gs0, gs1, gs2)
    ssem = (ss0, ss1, ss2)
    c = lax.axis_index("c")
    s = lax.axis_index("s")
    w = c * NS + s

    pltpu.sync_copy(dinv_hbm, dinv_v)
    pltpu.sync_copy(row_hbm.at[w], idxr_v)
    pltpu.sync_copy(col_hbm.at[w], idxc_v)
    pltpu.sync_copy(ew_hbm.at[w], ew_v)
    pltpu.sync_copy(tab_hbm.at[pl.ds(s * PT, PT)],
                    tab_s.at[pl.ds(s * PT, PT)])
    _zero_acc_slice(zero_v, acc, s, H2)
    plsc.subcore_barrier()

    _ring_agg(H2, tab_s, idxr_v, idxc_v, ew_v, dinv_v, grow, msg,
              gsem, ssem, acc)
    plsc.subcore_barrier()
    pltpu.sync_copy(acc.at[pl.ds(s * PT, PT)],
                    out_hbm.at[c, pl.ds(s * PT, PT)])


_agg2_call = pl.kernel(
    _agg2_body,
    out_type=jax.ShapeDtypeStruct((NC, NPAD, H2), jnp.float32),
    mesh=_MESH,
    compiler_params=_SC_PARAMS,
    scratch_types=[
        pltpu.VMEM((NPAD,), jnp.float32),
        pltpu.VMEM((SB, CH), jnp.int32),
        pltpu.VMEM((SB, CH), jnp.int32),
        pltpu.VMEM((SB, CH), jnp.float32),
        pltpu.VMEM((CH, H2), jnp.float32),
        pltpu.VMEM((CH, H2), jnp.float32),
        pltpu.VMEM((CH, H2), jnp.float32),
        pltpu.VMEM((CH, H2), jnp.float32),
        pltpu.VMEM((CH, H2), jnp.float32),
        pltpu.VMEM((CH, H2), jnp.float32),
        pltpu.VMEM((PT, H2), jnp.float32),
        pltpu.VMEM_SHARED((NPAD, H2), jnp.float32),
        pltpu.VMEM_SHARED((NPAD, H2), jnp.float32),
        pltpu.SemaphoreType.DMA,
        pltpu.SemaphoreType.DMA,
        pltpu.SemaphoreType.DMA,
        pltpu.SemaphoreType.DMA,
        pltpu.SemaphoreType.DMA,
        pltpu.SemaphoreType.DMA,
    ],
)


def _lin1_body(x_ref, w_ref, o_ref):
    o_ref[...] = jnp.dot(x_ref[...], w_ref[...],
                         preferred_element_type=jnp.float32)


def _relu_lin_body(p_ref, b_ref, w2_ref, o_ref):
    h = jnp.maximum(p_ref[0] + p_ref[1] + b_ref[...], 0.0)
    o_ref[...] = jnp.dot(h, w2_ref[...], preferred_element_type=jnp.float32)


def _head_body(q_ref, b2_ref, wfc_ref, bfc_ref, o_ref):
    m = q_ref[0] + q_ref[1] + b2_ref[...]
    o_ref[...] = jnp.dot(m, wfc_ref[...],
                         preferred_element_type=jnp.float32) + bfc_ref[...]


def kernel(x, edge_index, edge_weight, W1, b1, W2, b2, Wfc, bfc):
    n = x.shape[0]
    loop = jnp.arange(n, dtype=edge_index.dtype)
    row = jnp.concatenate([edge_index[0], loop])
    col = jnp.concatenate([edge_index[1], loop])
    ew = jnp.concatenate([edge_weight, jnp.ones((n,), edge_weight.dtype)])
    pad = EPAD - ETOT
    row3 = jnp.pad(row, (0, pad)).reshape(NW, SB, CH).astype(jnp.int32)
    col3 = jnp.pad(col, (0, pad)).reshape(NW, SB, CH).astype(jnp.int32)
    ew3 = jnp.pad(ew, (0, pad)).reshape(NW, SB, CH)
    xpad = jnp.pad(x, ((0, NPAD - n), (0, 0)))

    xw1 = pl.pallas_call(
        _lin1_body,
        out_shape=jax.ShapeDtypeStruct((NPAD, H1), jnp.float32),
    )(xpad, W1)

    p1, dinv = _agg1_call(xw1, row3, col3, ew3)

    xw2 = pl.pallas_call(
        _relu_lin_body,
        out_shape=jax.ShapeDtypeStruct((NPAD, H2), jnp.float32),
    )(p1, b1.reshape(1, H1), W2)

    p2 = _agg2_call(xw2, dinv, row3, col3, ew3)

    out = pl.pallas_call(
        _head_body,
        out_shape=jax.ShapeDtypeStruct((NPAD, 1), jnp.float32),
    )(p2, b2.reshape(1, H2), Wfc, bfc.reshape(1, 1))
    return out[:n]
